# Initial kernel scaffold; baseline (speedup 1.0000x reference)
import jax
import jax.numpy as jnp
from jax.experimental import pallas as pl


def kernel(observation, action, skill_idx, W1, b1, W2, b2, W3, b3):
    raise NotImplementedError("write your pallas kernel here")


# speedup vs baseline: 9.2265x; 9.2265x over previous
import jax
import jax.numpy as jnp
from jax.experimental import pallas as pl

_NUM_SKILLS = 16


def _mlp_body(x_ref, skill_ref, w1_ref, b1_ref, w2_ref, b2_ref, w3_ref,
              b3_ref, out_ref):
    e = pl.program_id(0)
    h = jnp.dot(x_ref[...], w1_ref[0], preferred_element_type=jnp.float32)
    h = jnp.maximum(h + b1_ref[0], 0.0)
    h = jnp.dot(h, w2_ref[0], preferred_element_type=jnp.float32)
    h = jnp.maximum(h + b2_ref[0], 0.0)
    logit = jnp.sum(h * w3_ref[0], axis=1, keepdims=True)
    prob = jax.nn.sigmoid(logit + b3_ref[0])
    contrib = jnp.where(skill_ref[...] == e, prob, 0.0)

    @pl.when(e == 0)
    def _init():
        out_ref[...] = contrib

    @pl.when(e > 0)
    def _acc():
        out_ref[...] = out_ref[...] + contrib


def kernel(observation, action, skill_idx, W1, b1, W2, b2, W3, b3):
    batch = observation.shape[0]
    in_dim = observation.shape[1] + action.shape[1]
    h1 = W1.shape[2]
    h2 = W2.shape[2]

    x = jnp.concatenate([observation, action], axis=1)
    skill = skill_idx.astype(jnp.int32).reshape(batch, 1)
    b1r = b1.reshape(_NUM_SKILLS, 1, h1)
    b2r = b2.reshape(_NUM_SKILLS, 1, h2)
    w3 = W3.reshape(_NUM_SKILLS, 1, h2)
    b3r = b3.reshape(_NUM_SKILLS, 1, 1)

    out = pl.pallas_call(
        _mlp_body,
        grid=(_NUM_SKILLS,),
        in_specs=[
            pl.BlockSpec((batch, in_dim), lambda e: (0, 0)),
            pl.BlockSpec((batch, 1), lambda e: (0, 0)),
            pl.BlockSpec((1, in_dim, h1), lambda e: (e, 0, 0)),
            pl.BlockSpec((1, 1, h1), lambda e: (e, 0, 0)),
            pl.BlockSpec((1, h1, h2), lambda e: (e, 0, 0)),
            pl.BlockSpec((1, 1, h2), lambda e: (e, 0, 0)),
            pl.BlockSpec((1, 1, h2), lambda e: (e, 0, 0)),
            pl.BlockSpec((1, 1, 1), lambda e: (e, 0, 0)),
        ],
        out_specs=pl.BlockSpec((batch, 1), lambda e: (0, 0)),
        out_shape=jax.ShapeDtypeStruct((batch, 1), jnp.float32),
    )(x, skill, W1, b1r, W2, b2r, w3, b3r)
    return out
